# baseline (speedup 1.0000x reference)
import functools

import numpy as np
import jax
import jax.numpy as jnp
from jax import lax
from jax.experimental import pallas as pl
from jax.experimental.pallas import tpu as pltpu
from jax.experimental.pallas import tpu_sc as plsc

_T = 8192
_NS = 16
_MEM = 4
_GAMMA = 0.5
_NSUB = 16
_STEPS = _T // _NSUB
_NCHUNK = 4
_CG = _T // _NS // _NCHUNK
_CW = _T * 8 // _NCHUNK


def _state_priors() -> np.ndarray:
    h = np.reshape(np.exp(-_GAMMA * np.arange(_MEM)), [1, _MEM])
    bits = np.unpackbits(
        np.arange(_NS).astype(np.uint8).reshape(-1, 1), axis=1
    ).astype(int)
    symbols = 1 - 2 * bits[:, -_MEM:]
    return np.dot(symbols, h.T).reshape(-1).astype(np.float32)


_SP = _state_priors()


@functools.cache
def _build_va_scan():
    return pl.kernel(
        _va_scan_body,
        out_type=jax.ShapeDtypeStruct((_T,), jnp.float32),
        mesh=plsc.VectorSubcoreMesh(core_axis_name="c", subcore_axis_name="s",
                                    num_cores=1),
        scratch_types=[
            pltpu.VMEM((_T,), jnp.float32),
            pltpu.VMEM((_NS,), jnp.float32),
            pltpu.VMEM((_T * 8,), jnp.float32),
            pltpu.VMEM((_STEPS * 8,), jnp.float32),
            pltpu.VMEM((_STEPS,), jnp.float32),
            pltpu.VMEM((_NS,), jnp.float32),
            pltpu.VMEM_SHARED((_T * 8,), jnp.float32),
            pltpu.SemaphoreType.DMA,
        ],
    )


def _va_scan_body(y_hbm, sp_hbm, out_hbm, y_v, sp_v, met_v, slice_v, bits_v,
                  p_v, met_sh, dma_sem):
    cid = lax.axis_index("c")
    sid = lax.axis_index("s")
    lanes = lax.broadcasted_iota(jnp.int32, (_NS,), 0)
    idx_e = (lanes & 7) * 2
    idx_o = idx_e + 1
    low = lanes < 8
    rslice = (sid + _NSUB - 1) % _NSUB

    @pl.when(cid == 0)
    def _():
        @pl.when(sid != 0)
        def _():
            pltpu.sync_copy(y_hbm.at[pl.ds(rslice * _STEPS, _STEPS)],
                            y_v.at[pl.ds(0, _STEPS)])
            pltpu.sync_copy(sp_hbm, sp_v)

        @pl.when(sid == 0)
        def _():
            pltpu.sync_copy(y_hbm, y_v)
            pltpu.sync_copy(sp_hbm, sp_v)
            p_v[...] = jnp.zeros((_NS,), jnp.float32)

        for chunk in range(_NCHUNK):
            @pl.when(sid == 0)
            def _():
                spv = sp_v[...]
                spe = spv.at[idx_e].get(mode="promise_in_bounds")
                spo = spv.at[idx_o].get(mode="promise_in_bounds")
                idx_bg = [
                    [(4 * lanes + 2 * b + gg) & 15 for gg in (0, 1)]
                    for b in (0, 1)
                ]

                def outer(g, p):
                    yv = y_v[pl.ds(g * _NS, _NS)]
                    for k in range(_NS // 2):
                        met_v[pl.ds(g * 128 + k * _NS, _NS)] = p
                        y0 = yv[2 * k]
                        y1 = yv[2 * k + 1]
                        v0 = p + jnp.abs(y0 - spv)
                        b0 = jnp.abs(y1 - spe)
                        b1 = jnp.abs(y1 - spo)
                        g00 = v0.at[idx_bg[0][0]].get(mode="promise_in_bounds")
                        g01 = v0.at[idx_bg[0][1]].get(mode="promise_in_bounds")
                        g10 = v0.at[idx_bg[1][0]].get(mode="promise_in_bounds")
                        g11 = v0.at[idx_bg[1][1]].get(mode="promise_in_bounds")
                        p = jnp.minimum(
                            jnp.minimum(g00 + b0, g01 + b0),
                            jnp.minimum(g10 + b1, g11 + b1),
                        )
                    return p

                pf = lax.fori_loop(chunk * _CG, (chunk + 1) * _CG, outer,
                                   p_v[...])
                p_v[...] = pf
                if chunk > 0:
                    pltpu.make_async_copy(
                        met_v.at[pl.ds((chunk - 1) * _CW, _CW)],
                        met_sh.at[pl.ds((chunk - 1) * _CW, _CW)],
                        dma_sem,
                    ).wait()

            plsc.subcore_barrier()

            @pl.when(sid == 0)
            def _():
                pltpu.async_copy(
                    met_v.at[pl.ds(chunk * _CW, _CW)],
                    met_sh.at[pl.ds(chunk * _CW, _CW)],
                    dma_sem,
                )

            if chunk > 0:
                _decode_chunk(chunk - 1, rslice, lanes, idx_e, idx_o, low,
                              y_v, sp_v, slice_v, bits_v, met_sh, out_hbm)

        @pl.when(sid == 0)
        def _():
            pltpu.make_async_copy(
                met_v.at[pl.ds((_NCHUNK - 1) * _CW, _CW)],
                met_sh.at[pl.ds((_NCHUNK - 1) * _CW, _CW)],
                dma_sem,
            ).wait()
        plsc.subcore_barrier()
        _decode_chunk(_NCHUNK - 1, rslice, lanes, idx_e, idx_o, low,
                      y_v, sp_v, slice_v, bits_v, met_sh, out_hbm)


def _decode_chunk(chunk, rslice, lanes, idx_e, idx_o, low,
                  y_v, sp_v, slice_v, bits_v, met_sh, out_hbm):
    slices_per_chunk = _NSUB // _NCHUNK

    @pl.when(rslice // slices_per_chunk == chunk)
    def _():
        pltpu.sync_copy(met_sh.at[pl.ds(rslice * (_STEPS * 8), _STEPS * 8)],
                        slice_v)
        sid = lax.axis_index("s")
        ybase = jnp.where(sid == 0, rslice * _STEPS, 0)
        spv = sp_v[...]
        half = lanes & 7

        def bfly_min(v):
            for d in (4, 2, 1):
                v = jnp.minimum(v, v.at[lanes ^ d].get(mode="promise_in_bounds"))
            return v

        def group(g, _):
            acc = jnp.zeros((_NS,), jnp.float32)
            yv = y_v[pl.ds(ybase + g * _NS, _NS)]
            for k in range(8):
                p0 = slice_v[pl.ds(g * 128 + k * _NS, _NS)]
                y0 = yv[2 * k]
                v0 = p0 + jnp.abs(y0 - spv)
                p1 = jnp.minimum(
                    v0.at[idx_e].get(mode="promise_in_bounds"),
                    v0.at[idx_o].get(mode="promise_in_bounds"),
                )
                v = jnp.where(low, p0, p1)
                m = bfly_min(v)
                cand = jnp.where(v == m, half, 8)
                idx = bfly_min(cand)
                bit = (idx % 2).astype(jnp.float32)
                acc = jnp.where(lanes == 2 * k, bit[0], acc)
                acc = jnp.where(lanes == 2 * k + 1, bit[8], acc)
            bits_v[pl.ds(g * _NS, _NS)] = acc
            return _

        lax.fori_loop(0, _STEPS // _NS, group, 0)
        pltpu.sync_copy(bits_v, out_hbm.at[pl.ds(rslice * _STEPS, _STEPS)])


def kernel(y):
    return _build_va_scan()(y.reshape(_T), jnp.asarray(_SP))
